# baseline (speedup 1.0000x reference)
import functools

import jax
import jax.numpy as jnp
from jax import lax
from jax.experimental import pallas as pl
from jax.experimental.pallas import tpu as pltpu
from jax.experimental.pallas import tpu_sc as plsc

NUM_EXPERTS = 64
TOP_K = 8
LANES = 16
NUM_TILES = 32
NUM_CHUNKS = 8
CHUNK_TOKENS = 4096
MM_BLK = 1024
BLK = CHUNK_TOKENS // NUM_TILES


def _logits_body(x_ref, w_ref, b_ref, out_ref, z_ref):
    i = pl.program_id(0)
    lt = jax.lax.dot_general(
        w_ref[...], x_ref[0], (((1,), (1,)), ((), ())),
        preferred_element_type=jnp.float32,
    ) + b_ref[...][:, None]
    out_ref[0] = lt
    zpart = jnp.sum(lt * lt).reshape(1, 1)

    @pl.when(i == 0)
    def _():
        z_ref[...] = zpart

    @pl.when(i > 0)
    def _():
        z_ref[...] += zpart


def _router_logits_t(x_chunked, W, b, chunk):
    _, chunk_tokens, hidden = x_chunked.shape
    grid = chunk_tokens // MM_BLK
    return pl.pallas_call(
        _logits_body,
        grid=(grid,),
        in_specs=[
            pl.BlockSpec((1, MM_BLK, hidden), lambda i, c=chunk: (c, i, 0)),
            pl.BlockSpec((NUM_EXPERTS, hidden), lambda i: (0, 0)),
            pl.BlockSpec((NUM_EXPERTS,), lambda i: (0,)),
        ],
        out_specs=[
            pl.BlockSpec((1, NUM_EXPERTS, MM_BLK), lambda i: (i, 0, 0)),
            pl.BlockSpec((1, 1), lambda i: (0, 0)),
        ],
        out_shape=[
            jax.ShapeDtypeStruct((grid, NUM_EXPERTS, MM_BLK), jnp.float32),
            jax.ShapeDtypeStruct((1, 1), jnp.float32),
        ],
    )(x_chunked, W, b)


def _routing_body(chunk, logits_hbm, cw_hbm, ei_hbm, hist_hbm,
                  buf, cwv, eiv, hist):
    info = plsc.get_sparse_core_info()
    nc = info.num_cores
    wid = lax.axis_index("s") * nc + lax.axis_index("c")
    blk_per_mm = MM_BLK // BLK
    mm_b = wid // blk_per_mm
    mm_off = (wid % blk_per_mm) * BLK

    pltpu.sync_copy(logits_hbm.at[mm_b, :, pl.ds(mm_off, BLK)], buf)

    lane = lax.iota(jnp.int32, 16)
    neg_inf = jnp.full((16,), -jnp.inf, jnp.float32)
    zero_i = jnp.zeros((16,), jnp.int32)
    one_i = jnp.ones((16,), jnp.int32)

    def init_hist(i, c):
        hist[pl.ds(i * 16, 16)] = zero_i
        return c

    lax.fori_loop(0, NUM_EXPERTS, init_hist, 0, unroll=8)

    def group_body(g, carry):
        col = g * 16
        tokv = col + lane

        def max_body(e, m):
            return jnp.maximum(m, buf[e, pl.ds(col, 16)])

        m = lax.fori_loop(0, NUM_EXPERTS, max_body, neg_inf, unroll=8)

        def exp_body(e, s):
            return s + jnp.exp(buf[e, pl.ds(col, 16)] - m)

        s = lax.fori_loop(0, NUM_EXPERTS, exp_body,
                          jnp.zeros((16,), jnp.float32), unroll=8)
        rcp = 1.0 / s

        def k_body(k, carry2):
            def am_body(e, mi):
                m2, idx = mi
                v = buf[e, pl.ds(col, 16)]
                pred = v > m2
                return jnp.maximum(m2, v), jnp.where(pred, e, idx)

            m2, idx = lax.fori_loop(0, NUM_EXPERTS, am_body,
                                    (neg_inf, zero_i), unroll=8)
            w = jnp.exp(m2 - m) * rcp
            img_flat = tokv * 128 + k
            plsc.store_scatter(cwv, [img_flat], w)
            plsc.store_scatter(eiv, [img_flat], idx)
            plsc.addupdate_scatter(hist, [idx * 16 + lane], one_i)
            plsc.store_scatter(buf, [idx, tokv], neg_inf)
            return carry2

        lax.fori_loop(0, TOP_K, k_body, 0)
        return carry

    lax.fori_loop(0, BLK // 16, group_body, 0)

    pltpu.sync_copy(cwv, cw_hbm.at[pl.ds(wid * BLK * 128, BLK * 128)])
    pltpu.sync_copy(eiv, ei_hbm.at[pl.ds(wid * BLK * 128, BLK * 128)])
    pltpu.sync_copy(hist, hist_hbm.at[wid])


def _routing(chunk, logits_t):
    mesh = plsc.VectorSubcoreMesh(core_axis_name="c", subcore_axis_name="s")
    return pl.kernel(
        functools.partial(_routing_body, chunk),
        out_type=[
            jax.ShapeDtypeStruct((CHUNK_TOKENS * 128,), jnp.float32),
            jax.ShapeDtypeStruct((CHUNK_TOKENS * 128,), jnp.int32),
            jax.ShapeDtypeStruct((NUM_TILES, NUM_EXPERTS * LANES), jnp.int32),
        ],
        mesh=mesh,
        compiler_params=pltpu.CompilerParams(needs_layout_passes=False),
        scratch_types=[
            pltpu.VMEM((NUM_EXPERTS, BLK), jnp.float32),
            pltpu.VMEM((BLK * 128,), jnp.float32),
            pltpu.VMEM((BLK * 128,), jnp.int32),
            pltpu.VMEM((NUM_EXPERTS * LANES,), jnp.int32),
        ],
    )(logits_t)


def kernel(token_inputs, expert_capacity, W, b):
    num_groups, tokens_per_group, hidden_dim = token_inputs.shape
    num_experts = W.shape[0]
    batch_size = num_groups * tokens_per_group

    x_chunked = token_inputs.reshape(NUM_CHUNKS, CHUNK_TOKENS, hidden_dim)

    cw_l, ei_l, hist_l, z_l = [], [], [], []
    for c in range(NUM_CHUNKS):
        logits_t, z_c = _router_logits_t(x_chunked, W, b, c)
        cw, ei, hist = _routing(c, logits_t)
        cw_l.append(cw)
        ei_l.append(ei)
        hist_l.append(hist)
        z_l.append(z_c)

    tokens_per_expert = jnp.stack(hist_l).reshape(
        -1, num_experts, LANES).astype(jnp.float32).sum(axis=(0, 2))
    target = tokens_per_expert.sum() / num_experts
    auxiliary_loss = jnp.mean((tokens_per_expert - target) ** 2)

    z_sum = sum(z[0, 0] for z in z_l)
    router_z_loss = z_sum / (batch_size * num_experts)

    combine_weights = jnp.stack(
        [c.reshape(CHUNK_TOKENS, 128)[:, :TOP_K] for c in cw_l]).reshape(
            num_groups, tokens_per_group, TOP_K)
    expert_indices = jnp.stack(
        [e.reshape(CHUNK_TOKENS, 128)[:, :TOP_K] for e in ei_l]).reshape(
            num_groups, tokens_per_group, TOP_K)
    batch_ids = jax.lax.broadcasted_iota(
        jnp.int32, (num_groups, tokens_per_group, TOP_K), 1) + (
            jnp.arange(num_groups, dtype=jnp.int32)[:, None, None]
            * tokens_per_group)
    dispatch_indices = jnp.stack([batch_ids, expert_indices], axis=-1)
    return (dispatch_indices, combine_weights, auxiliary_loss, router_z_loss)


# speedup vs baseline: 1.1057x; 1.1057x over previous
import functools

import jax
import jax.numpy as jnp
from jax import lax
from jax.experimental import pallas as pl
from jax.experimental.pallas import tpu as pltpu
from jax.experimental.pallas import tpu_sc as plsc

NUM_EXPERTS = 64
TOP_K = 8
LANES = 16
NUM_TILES = 32
NUM_CHUNKS = 4
CHUNK_TOKENS = 8192
MM_BLK = 1024
BLK = CHUNK_TOKENS // NUM_TILES


def _logits_body(x_ref, w_ref, b_ref, out_ref, z_ref):
    i = pl.program_id(0)
    lt = jax.lax.dot_general(
        w_ref[...], x_ref[0], (((1,), (1,)), ((), ())),
        preferred_element_type=jnp.float32,
    ) + b_ref[...][:, None]
    out_ref[0] = lt
    zpart = jnp.sum(lt * lt).reshape(1, 1)

    @pl.when(i == 0)
    def _():
        z_ref[...] = zpart

    @pl.when(i > 0)
    def _():
        z_ref[...] += zpart


def _router_logits_t(x_chunked, W, b, chunk):
    _, chunk_tokens, hidden = x_chunked.shape
    grid = chunk_tokens // MM_BLK
    return pl.pallas_call(
        _logits_body,
        grid=(grid,),
        in_specs=[
            pl.BlockSpec((1, MM_BLK, hidden), lambda i, c=chunk: (c, i, 0)),
            pl.BlockSpec((NUM_EXPERTS, hidden), lambda i: (0, 0)),
            pl.BlockSpec((NUM_EXPERTS,), lambda i: (0,)),
        ],
        out_specs=[
            pl.BlockSpec((1, NUM_EXPERTS, MM_BLK), lambda i: (i, 0, 0)),
            pl.BlockSpec((1, 1), lambda i: (0, 0)),
        ],
        out_shape=[
            jax.ShapeDtypeStruct((grid, NUM_EXPERTS, MM_BLK), jnp.float32),
            jax.ShapeDtypeStruct((1, 1), jnp.float32),
        ],
    )(x_chunked, W, b)


def _routing_body(logits_hbm, cw_hbm, ei_hbm, hist_hbm,
                  buf, cwv, eiv, hist):
    info = plsc.get_sparse_core_info()
    nc = info.num_cores
    wid = lax.axis_index("s") * nc + lax.axis_index("c")
    blk_per_mm = MM_BLK // BLK
    mm_b = wid // blk_per_mm
    mm_off = (wid % blk_per_mm) * BLK

    pltpu.sync_copy(logits_hbm.at[mm_b, :, pl.ds(mm_off, BLK)], buf)

    lane = lax.iota(jnp.int32, 16)
    neg_inf = jnp.full((16,), -jnp.inf, jnp.float32)
    zero_i = jnp.zeros((16,), jnp.int32)
    one_i = jnp.ones((16,), jnp.int32)

    def init_hist(i, c):
        hist[pl.ds(i * 16, 16)] = zero_i
        return c

    lax.fori_loop(0, NUM_EXPERTS, init_hist, 0, unroll=8)

    def group_body(g, carry):
        col = g * 16
        tokv = col + lane

        def max_body(e, m):
            return jnp.maximum(m, buf[e, pl.ds(col, 16)])

        m = lax.fori_loop(0, NUM_EXPERTS, max_body, neg_inf, unroll=8)

        def exp_body(e, s):
            return s + jnp.exp(buf[e, pl.ds(col, 16)] - m)

        s = lax.fori_loop(0, NUM_EXPERTS, exp_body,
                          jnp.zeros((16,), jnp.float32), unroll=8)
        rcp = 1.0 / s

        def k_body(k, carry2):
            def am_body(e, mi):
                m2, idx = mi
                v = buf[e, pl.ds(col, 16)]
                pred = v > m2
                return jnp.maximum(m2, v), jnp.where(pred, e, idx)

            m2, idx = lax.fori_loop(0, NUM_EXPERTS, am_body,
                                    (neg_inf, zero_i), unroll=8)
            w = jnp.exp(m2 - m) * rcp
            img_flat = tokv * 128 + k
            plsc.store_scatter(cwv, [img_flat], w)
            plsc.store_scatter(eiv, [img_flat], idx)
            plsc.addupdate_scatter(hist, [idx * 16 + lane], one_i)
            plsc.store_scatter(buf, [idx, tokv], neg_inf)
            return carry2

        lax.fori_loop(0, TOP_K, k_body, 0)
        return carry

    lax.fori_loop(0, BLK // 16, group_body, 0)

    pltpu.sync_copy(cwv, cw_hbm.at[pl.ds(wid * BLK * 128, BLK * 128)])
    pltpu.sync_copy(eiv, ei_hbm.at[pl.ds(wid * BLK * 128, BLK * 128)])
    pltpu.sync_copy(hist, hist_hbm.at[wid])


_routing_cache = {}


def _routing(logits_t):
    if "k" not in _routing_cache:
        mesh = plsc.VectorSubcoreMesh(
            core_axis_name="c", subcore_axis_name="s")
        _routing_cache["k"] = _make_routing_kernel(mesh)
    return _routing_cache["k"](logits_t)


def _make_routing_kernel(mesh):
    return pl.kernel(
        _routing_body,
        out_type=[
            jax.ShapeDtypeStruct((CHUNK_TOKENS * 128,), jnp.float32),
            jax.ShapeDtypeStruct((CHUNK_TOKENS * 128,), jnp.int32),
            jax.ShapeDtypeStruct((NUM_TILES, NUM_EXPERTS * LANES), jnp.int32),
        ],
        mesh=mesh,
        compiler_params=pltpu.CompilerParams(needs_layout_passes=False),
        scratch_types=[
            pltpu.VMEM((NUM_EXPERTS, BLK), jnp.float32),
            pltpu.VMEM((BLK * 128,), jnp.float32),
            pltpu.VMEM((BLK * 128,), jnp.int32),
            pltpu.VMEM((NUM_EXPERTS * LANES,), jnp.int32),
        ],
    )


def kernel(token_inputs, expert_capacity, W, b):
    num_groups, tokens_per_group, hidden_dim = token_inputs.shape
    num_experts = W.shape[0]
    batch_size = num_groups * tokens_per_group

    x_chunked = token_inputs.reshape(NUM_CHUNKS, CHUNK_TOKENS, hidden_dim)

    cw_l, ei_l, hist_l, z_l = [], [], [], []
    for c in range(NUM_CHUNKS):
        logits_t, z_c = _router_logits_t(x_chunked, W, b, c)
        cw, ei, hist = _routing(logits_t)
        cw_l.append(cw)
        ei_l.append(ei)
        hist_l.append(hist)
        z_l.append(z_c)

    tokens_per_expert = jnp.stack(hist_l).reshape(
        -1, num_experts, LANES).astype(jnp.float32).sum(axis=(0, 2))
    target = tokens_per_expert.sum() / num_experts
    auxiliary_loss = jnp.mean((tokens_per_expert - target) ** 2)

    z_sum = sum(z[0, 0] for z in z_l)
    router_z_loss = z_sum / (batch_size * num_experts)

    combine_weights = jnp.stack(
        [c.reshape(CHUNK_TOKENS, 128)[:, :TOP_K] for c in cw_l]).reshape(
            num_groups, tokens_per_group, TOP_K)
    expert_indices = jnp.stack(
        [e.reshape(CHUNK_TOKENS, 128)[:, :TOP_K] for e in ei_l]).reshape(
            num_groups, tokens_per_group, TOP_K)
    batch_ids = jax.lax.broadcasted_iota(
        jnp.int32, (num_groups, tokens_per_group, TOP_K), 1) + (
            jnp.arange(num_groups, dtype=jnp.int32)[:, None, None]
            * tokens_per_group)
    dispatch_indices = jnp.stack([batch_ids, expert_indices], axis=-1)
    return (dispatch_indices, combine_weights, auxiliary_loss, router_z_loss)


# speedup vs baseline: 1.1425x; 1.0332x over previous
import functools

import jax
import jax.numpy as jnp
from jax import lax
from jax.experimental import pallas as pl
from jax.experimental.pallas import tpu as pltpu
from jax.experimental.pallas import tpu_sc as plsc

NUM_EXPERTS = 64
TOP_K = 8
LANES = 16
NUM_TILES = 32
CHUNK_SIZES = (12288, 12288, 4096, 4096)
MM_BLK = 1024


def _logits_body(x_ref, w_ref, b_ref, out_ref, z_ref):
    i = pl.program_id(0)
    lt = jax.lax.dot_general(
        w_ref[...], x_ref[0], (((1,), (1,)), ((), ())),
        preferred_element_type=jnp.float32,
    ) + b_ref[...][:, None]
    out_ref[...] = lt
    zpart = jnp.sum(lt * lt).reshape(1, 1)

    @pl.when(i == 0)
    def _():
        z_ref[...] = zpart

    @pl.when(i > 0)
    def _():
        z_ref[...] += zpart


def _router_logits_t(x_blocked, W, b, off_blk, n_blk):
    hidden = x_blocked.shape[-1]
    return pl.pallas_call(
        _logits_body,
        grid=(n_blk,),
        in_specs=[
            pl.BlockSpec((1, MM_BLK, hidden),
                         lambda i, o=off_blk: (o + i, 0, 0)),
            pl.BlockSpec((NUM_EXPERTS, hidden), lambda i: (0, 0)),
            pl.BlockSpec((NUM_EXPERTS,), lambda i: (0,)),
        ],
        out_specs=[
            pl.BlockSpec((NUM_EXPERTS, MM_BLK), lambda i: (0, i)),
            pl.BlockSpec((1, 1), lambda i: (0, 0)),
        ],
        out_shape=[
            jax.ShapeDtypeStruct((NUM_EXPERTS, n_blk * MM_BLK), jnp.float32),
            jax.ShapeDtypeStruct((1, 1), jnp.float32),
        ],
    )(x_blocked, W, b)


def _routing_body(logits_hbm, cw_hbm, ei_hbm, hist_hbm,
                  buf, cwv, eiv, hist):
    blk = buf.shape[1]
    info = plsc.get_sparse_core_info()
    nc = info.num_cores
    wid = lax.axis_index("s") * nc + lax.axis_index("c")

    pltpu.sync_copy(logits_hbm.at[:, pl.ds(wid * blk, blk)], buf)

    lane = lax.iota(jnp.int32, 16)
    neg_inf = jnp.full((16,), -jnp.inf, jnp.float32)
    zero_i = jnp.zeros((16,), jnp.int32)
    one_i = jnp.ones((16,), jnp.int32)

    def init_hist(i, c):
        hist[pl.ds(i * 16, 16)] = zero_i
        return c

    lax.fori_loop(0, NUM_EXPERTS, init_hist, 0, unroll=8)

    def group_body(g, carry):
        col = g * 16
        tokv = col + lane

        def max_body(e, m):
            return jnp.maximum(m, buf[e, pl.ds(col, 16)])

        m = lax.fori_loop(0, NUM_EXPERTS, max_body, neg_inf, unroll=8)

        def exp_body(e, s):
            return s + jnp.exp(buf[e, pl.ds(col, 16)] - m)

        s = lax.fori_loop(0, NUM_EXPERTS, exp_body,
                          jnp.zeros((16,), jnp.float32), unroll=8)
        rcp = 1.0 / s

        def k_body(k, carry2):
            def am_body(e, mi):
                m2, idx = mi
                v = buf[e, pl.ds(col, 16)]
                pred = v > m2
                return jnp.maximum(m2, v), jnp.where(pred, e, idx)

            m2, idx = lax.fori_loop(0, NUM_EXPERTS, am_body,
                                    (neg_inf, zero_i), unroll=8)
            w = jnp.exp(m2 - m) * rcp
            img_flat = tokv * 128 + k
            plsc.store_scatter(cwv, [img_flat], w)
            plsc.store_scatter(eiv, [img_flat], idx)
            plsc.addupdate_scatter(hist, [idx * 16 + lane], one_i)
            plsc.store_scatter(buf, [idx, tokv], neg_inf)
            return carry2

        lax.fori_loop(0, TOP_K, k_body, 0)
        return carry

    lax.fori_loop(0, blk // 16, group_body, 0)

    pltpu.sync_copy(cwv, cw_hbm.at[pl.ds(wid * blk * 128, blk * 128)])
    pltpu.sync_copy(eiv, ei_hbm.at[pl.ds(wid * blk * 128, blk * 128)])
    pltpu.sync_copy(hist, hist_hbm.at[wid])


_routing_cache = {}


def _routing(logits_t):
    chunk_tokens = logits_t.shape[1]
    if chunk_tokens not in _routing_cache:
        mesh = plsc.VectorSubcoreMesh(
            core_axis_name="c", subcore_axis_name="s")
        _routing_cache[chunk_tokens] = _make_routing_kernel(
            mesh, chunk_tokens)
    return _routing_cache[chunk_tokens](logits_t)


def _make_routing_kernel(mesh, chunk_tokens):
    blk = chunk_tokens // NUM_TILES
    return pl.kernel(
        _routing_body,
        out_type=[
            jax.ShapeDtypeStruct((chunk_tokens * 128,), jnp.float32),
            jax.ShapeDtypeStruct((chunk_tokens * 128,), jnp.int32),
            jax.ShapeDtypeStruct((NUM_TILES, NUM_EXPERTS * LANES), jnp.int32),
        ],
        mesh=mesh,
        compiler_params=pltpu.CompilerParams(needs_layout_passes=False),
        scratch_types=[
            pltpu.VMEM((NUM_EXPERTS, blk), jnp.float32),
            pltpu.VMEM((blk * 128,), jnp.float32),
            pltpu.VMEM((blk * 128,), jnp.int32),
            pltpu.VMEM((NUM_EXPERTS * LANES,), jnp.int32),
        ],
    )


def kernel(token_inputs, expert_capacity, W, b):
    num_groups, tokens_per_group, hidden_dim = token_inputs.shape
    num_experts = W.shape[0]
    batch_size = num_groups * tokens_per_group

    x_blocked = token_inputs.reshape(-1, MM_BLK, hidden_dim)

    cw_l, ei_l, hist_l, z_l = [], [], [], []
    off_blk = 0
    for size in CHUNK_SIZES:
        n_blk = size // MM_BLK
        logits_t, z_c = _router_logits_t(x_blocked, W, b, off_blk, n_blk)
        off_blk += n_blk
        cw, ei, hist = _routing(logits_t)
        cw_l.append(cw)
        ei_l.append(ei)
        hist_l.append(hist)
        z_l.append(z_c)

    tokens_per_expert = jnp.stack(hist_l).reshape(
        -1, num_experts, LANES).astype(jnp.float32).sum(axis=(0, 2))
    target = tokens_per_expert.sum() / num_experts
    auxiliary_loss = jnp.mean((tokens_per_expert - target) ** 2)

    z_sum = sum(z[0, 0] for z in z_l)
    router_z_loss = z_sum / (batch_size * num_experts)

    combine_weights = jnp.concatenate(
        [c.reshape(-1, 128)[:, :TOP_K] for c in cw_l]).reshape(
            num_groups, tokens_per_group, TOP_K)
    expert_indices = jnp.concatenate(
        [e.reshape(-1, 128)[:, :TOP_K] for e in ei_l]).reshape(
            num_groups, tokens_per_group, TOP_K)
    batch_ids = jax.lax.broadcasted_iota(
        jnp.int32, (num_groups, tokens_per_group, TOP_K), 1) + (
            jnp.arange(num_groups, dtype=jnp.int32)[:, None, None]
            * tokens_per_group)
    dispatch_indices = jnp.stack([batch_ids, expert_indices], axis=-1)
    return (dispatch_indices, combine_weights, auxiliary_loss, router_z_loss)
